# Initial kernel scaffold; baseline (speedup 1.0000x reference)
import jax
import jax.numpy as jnp
from jax.experimental import pallas as pl


def kernel(x, edge_attr, y, edge_index, batch, params):
    raise NotImplementedError("write your pallas kernel here")


# speedup vs baseline: 1.0000x; 1.0000x over previous
import math

import jax
import jax.numpy as jnp
from jax.experimental import pallas as pl

EMBED = 24


def _gru(m, h, Wih, Whh, bih, bhh):
    gi = m @ Wih.T + bih
    gh = h @ Whh.T + bhh
    ir, iz, inn = jnp.split(gi, 3, axis=1)
    hr, hz, hn = jnp.split(gh, 3, axis=1)
    r = jax.nn.sigmoid(ir + hr)
    z = jax.nn.sigmoid(iz + hz)
    n = jnp.tanh(inn + r * hn)
    return (1.0 - z) * n + z * h


def _ggc(x, ei, ew, p, pre):
    src, dst = ei[0], ei[1]
    W = p[pre + '_W']
    for i in range(2):
        m = x @ W[i]
        msg = m[src] * ew[:, None]
        agg = jnp.zeros_like(x).at[dst].add(msg)
        x = _gru(agg, x, p[pre + '_Wih'], p[pre + '_Whh'], p[pre + '_bih'], p[pre + '_bhh'])
    return x


def _topk_pool(x, ei, ea, batch, w, Bn, n_per, ratio):
    k = math.ceil(n_per * ratio)
    score = jnp.tanh((x @ w) / jnp.linalg.norm(w))
    sv, si = jax.lax.top_k(score.reshape(Bn, n_per), k)
    perm = (si + (jnp.arange(Bn) * n_per)[:, None]).reshape(-1)
    xn = x[perm] * score[perm][:, None]
    bn = batch[perm]
    mapping = jnp.full((Bn * n_per,), -1, dtype=jnp.int32).at[perm].set(jnp.arange(Bn * k, dtype=jnp.int32))
    ns = mapping[ei[0]]
    nd = mapping[ei[1]]
    valid = (ns >= 0) & (nd >= 0)
    ns = jnp.where(valid, ns, 0)
    nd = jnp.where(valid, nd, 0)
    ean = jnp.where(valid[:, None], ea, 0.0)
    return xn, jnp.stack([ns, nd]), ean, bn


def _set2set(x, Bn, n_per, p):
    xr = x.reshape(Bn, n_per, EMBED)
    h = jnp.zeros((Bn, EMBED), dtype=x.dtype)
    c = jnp.zeros((Bn, EMBED), dtype=x.dtype)
    q_star = jnp.zeros((Bn, 2 * EMBED), dtype=x.dtype)
    for _ in range(2):
        g = q_star @ p['lstm_Wih'].T + p['lstm_bih'] + h @ p['lstm_Whh'].T + p['lstm_bhh']
        ii, ff, gg, oo = jnp.split(g, 4, axis=1)
        ii = jax.nn.sigmoid(ii)
        ff = jax.nn.sigmoid(ff)
        gg = jnp.tanh(gg)
        oo = jax.nn.sigmoid(oo)
        c = ff * c + ii * gg
        h = oo * jnp.tanh(c)
        q = h
        e = (xr * q[:, None, :]).sum(-1)
        a = jax.nn.softmax(e, axis=1)
        r = (a[..., None] * xr).sum(1)
        q_star = jnp.concatenate([q, r], axis=1)
    return q_star


def kernel(x, edge_attr, y, edge_index, batch, params):
    Bn = y.shape[0]
    n0 = x.shape[0] // Bn
    indices = jnp.tile(jnp.arange(n0), Bn)
    x = jax.nn.relu(_ggc(x, edge_index, edge_attr[:, 0], params, 'conv1'))
    x, ei, ea, batch = _topk_pool(x, edge_index, edge_attr, batch, params['pool1_w'], Bn, n0, 0.8)
    n1 = x.shape[0] // Bn
    x = jax.nn.relu(_ggc(x, ei, ea[:, 0], params, 'conv2'))
    x, ei, ea, batch = _topk_pool(x, ei, ea, batch, params['pool2_w'], Bn, n1, 0.8)
    n2 = x.shape[0] // Bn
    x = jax.nn.relu(_ggc(x, ei, ea[:, 0], params, 'conv3'))
    x, ei, ea, batch = _topk_pool(x, ei, ea, batch, params['pool3_w'], Bn, n2, 0.3)
    n3 = x.shape[0] // Bn
    x = jax.nn.relu(_ggc(x, ei, ea[:, 0], params, 'conv4'))
    x = jax.nn.relu(_ggc(x, ei, ea[:, 0], params, 'conv5'))
    x = jax.nn.relu(_ggc(x, ei, ea[:, 0], params, 'conv6'))
    xr = x.reshape(Bn, n3, EMBED)
    gmp = xr.max(axis=1)
    gap = xr.mean(axis=1)
    s2s = _set2set(x, Bn, n3, params)
    x6 = jnp.concatenate([gmp, gap, s2s], axis=1)
    out = jax.nn.relu(x6 @ params['lin1_W'].T + params['lin1_b'])
    return out, indices


# speedup vs baseline: 1.0718x; 1.0717x over previous
import functools
import math

import jax
import jax.numpy as jnp
from jax import lax
from jax.experimental import pallas as pl
from jax.experimental.pallas import tpu as pltpu
from jax.experimental.pallas import tpu_sc as plsc

EMBED = 24


_SC_C = 1024
_SC_ZR = 128


@functools.cache
def _msgpass_sc(NNp: int, nchunks: int):
    C = _SC_C
    ZR = _SC_ZR
    R = NNp // 16
    cpt = nchunks // 16
    assert NNp % (16 * ZR) == 0 and nchunks % 16 == 0
    mesh = plsc.VectorSubcoreMesh(core_axis_name="c", subcore_axis_name="s")

    @functools.partial(
        pl.kernel,
        out_type=jax.ShapeDtypeStruct((2, NNp, 16), jnp.float32),
        mesh=mesh,
        compiler_params=pltpu.CompilerParams(
            needs_layout_passes=False, use_tc_tiling_on_sc=False),
        scratch_types=[
            pltpu.VMEM_SHARED((NNp, 16), jnp.float32),
            pltpu.VMEM((C,), jnp.int32),
            pltpu.VMEM((C,), jnp.int32),
            pltpu.VMEM((C,), jnp.float32),
            pltpu.VMEM((C, 16), jnp.float32),
            pltpu.VMEM((ZR, 16), jnp.float32),
            pltpu.SemaphoreType.DMA,
        ],
    )
    def k(m2, srch, dsth, ewh, out, agg_sh, src_v, dst_v, ew_v, rows_v, zbuf_v, sem):
        c = lax.axis_index("c")
        s = lax.axis_index("s")

        def zb(i, carry):
            zbuf_v[i] = jnp.zeros((16,), jnp.float32)
            return carry
        lax.fori_loop(0, ZR, zb, 0)

        def za(i, carry):
            pltpu.sync_copy(zbuf_v, agg_sh.at[pl.ds(s * R + i * ZR, ZR)])
            return carry
        lax.fori_loop(0, R // ZR, za, 0)
        plsc.subcore_barrier()

        def chunk_body(t, carry):
            base = (t * 16 + s) * C
            pltpu.sync_copy(srch.at[pl.ds(base, C)], src_v)
            pltpu.sync_copy(dsth.at[pl.ds(base, C)], dst_v)
            pltpu.sync_copy(ewh.at[pl.ds(base, C)], ew_v)

            @pl.when(c == 0)
            def _():
                pltpu.async_copy(m2.at[0].at[src_v], rows_v, sem).wait()

            @pl.when(c == 1)
            def _():
                pltpu.async_copy(m2.at[1].at[src_v], rows_v, sem).wait()

            def mul_body(gi, carry2):
                eb = gi * 16
                for j in range(16):
                    w = plsc.load_gather(ew_v, [jnp.full((16,), eb + j, jnp.int32)])
                    rows_v[eb + j] = rows_v[eb + j] * w
                return carry2
            lax.fori_loop(0, C // 16, mul_body, 0)

            pltpu.sync_copy(rows_v, agg_sh.at[dst_v], add=True)
            return carry
        lax.fori_loop(0, cpt, chunk_body, 0)
        plsc.subcore_barrier()

        def wb(i, carry):
            rs = s * R + i * ZR
            pltpu.sync_copy(agg_sh.at[pl.ds(rs, ZR)], zbuf_v)

            @pl.when(c == 0)
            def _():
                pltpu.sync_copy(zbuf_v, out.at[0].at[pl.ds(rs, ZR)])

            @pl.when(c == 1)
            def _():
                pltpu.sync_copy(zbuf_v, out.at[1].at[pl.ds(rs, ZR)])
            return carry
        lax.fori_loop(0, R // ZR, wb, 0)

    return k


def _pad_to(n: int, mult: int) -> int:
    return ((n + mult - 1) // mult) * mult


def _sc_agg(m, srcp, dstp, ewp, NN):
    NNp = _pad_to(NN, 16 * _SC_ZR)
    nchunks = ewp.shape[0] // _SC_C
    m2 = jnp.zeros((2, NNp, 16), jnp.float32)
    m2 = m2.at[0, :NN, :12].set(m[:, :12])
    m2 = m2.at[1, :NN, :12].set(m[:, 12:])
    agg2 = _msgpass_sc(NNp, nchunks)(m2, srcp, dstp, ewp)
    return jnp.concatenate([agg2[0, :NN, :12], agg2[1, :NN, :12]], axis=1)


def _gru(m, h, Wih, Whh, bih, bhh):
    gi = m @ Wih.T + bih
    gh = h @ Whh.T + bhh
    ir, iz, inn = jnp.split(gi, 3, axis=1)
    hr, hz, hn = jnp.split(gh, 3, axis=1)
    r = jax.nn.sigmoid(ir + hr)
    z = jax.nn.sigmoid(iz + hz)
    n = jnp.tanh(inn + r * hn)
    return (1.0 - z) * n + z * h


def _ggc(x, ei, ew, p, pre):
    src, dst = ei[0], ei[1]
    NN = x.shape[0]
    W = p[pre + '_W']
    for i in range(2):
        m = x @ W[i]
        agg = _sc_agg(m, src, dst, ew, NN)
        x = _gru(agg, x, p[pre + '_Wih'], p[pre + '_Whh'], p[pre + '_bih'], p[pre + '_bhh'])
    return x


def _topk_pool(x, ei, ea, batch, w, Bn, n_per, ratio):
    k = math.ceil(n_per * ratio)
    score = jnp.tanh((x @ w) / jnp.linalg.norm(w))
    sv, si = jax.lax.top_k(score.reshape(Bn, n_per), k)
    perm = (si + (jnp.arange(Bn) * n_per)[:, None]).reshape(-1)
    xn = x[perm] * score[perm][:, None]
    bn = batch[perm]
    mapping = jnp.full((Bn * n_per,), -1, dtype=jnp.int32).at[perm].set(jnp.arange(Bn * k, dtype=jnp.int32))
    ns = mapping[ei[0]]
    nd = mapping[ei[1]]
    valid = (ns >= 0) & (nd >= 0)
    ns = jnp.where(valid, ns, 0)
    nd = jnp.where(valid, nd, 0)
    ean = jnp.where(valid[:, None], ea, 0.0)
    return xn, jnp.stack([ns, nd]), ean, bn


def _set2set(x, Bn, n_per, p):
    xr = x.reshape(Bn, n_per, EMBED)
    h = jnp.zeros((Bn, EMBED), dtype=x.dtype)
    c = jnp.zeros((Bn, EMBED), dtype=x.dtype)
    q_star = jnp.zeros((Bn, 2 * EMBED), dtype=x.dtype)
    for _ in range(2):
        g = q_star @ p['lstm_Wih'].T + p['lstm_bih'] + h @ p['lstm_Whh'].T + p['lstm_bhh']
        ii, ff, gg, oo = jnp.split(g, 4, axis=1)
        ii = jax.nn.sigmoid(ii)
        ff = jax.nn.sigmoid(ff)
        gg = jnp.tanh(gg)
        oo = jax.nn.sigmoid(oo)
        c = ff * c + ii * gg
        h = oo * jnp.tanh(c)
        q = h
        e = (xr * q[:, None, :]).sum(-1)
        a = jax.nn.softmax(e, axis=1)
        r = (a[..., None] * xr).sum(1)
        q_star = jnp.concatenate([q, r], axis=1)
    return q_star


def kernel(x, edge_attr, y, edge_index, batch, params):
    Bn = y.shape[0]
    n0 = x.shape[0] // Bn
    indices = jnp.tile(jnp.arange(n0), Bn)
    E = edge_index.shape[1]
    Ep = _pad_to(E, _SC_C * 16)
    edge_index = jnp.pad(edge_index, ((0, 0), (0, Ep - E)))
    edge_attr = jnp.pad(edge_attr, ((0, Ep - E), (0, 0)))
    x = jax.nn.relu(_ggc(x, edge_index, edge_attr[:, 0], params, 'conv1'))
    x, ei, ea, batch = _topk_pool(x, edge_index, edge_attr, batch, params['pool1_w'], Bn, n0, 0.8)
    n1 = x.shape[0] // Bn
    x = jax.nn.relu(_ggc(x, ei, ea[:, 0], params, 'conv2'))
    x, ei, ea, batch = _topk_pool(x, ei, ea, batch, params['pool2_w'], Bn, n1, 0.8)
    n2 = x.shape[0] // Bn
    x = jax.nn.relu(_ggc(x, ei, ea[:, 0], params, 'conv3'))
    x, ei, ea, batch = _topk_pool(x, ei, ea, batch, params['pool3_w'], Bn, n2, 0.3)
    n3 = x.shape[0] // Bn
    x = jax.nn.relu(_ggc(x, ei, ea[:, 0], params, 'conv4'))
    x = jax.nn.relu(_ggc(x, ei, ea[:, 0], params, 'conv5'))
    x = jax.nn.relu(_ggc(x, ei, ea[:, 0], params, 'conv6'))
    xr = x.reshape(Bn, n3, EMBED)
    gmp = xr.max(axis=1)
    gap = xr.mean(axis=1)
    s2s = _set2set(x, Bn, n3, params)
    x6 = jnp.concatenate([gmp, gap, s2s], axis=1)
    out = jax.nn.relu(x6 @ params['lin1_W'].T + params['lin1_b'])
    return out, indices


# speedup vs baseline: 1.5623x; 1.4577x over previous
import functools
import math

import jax
import jax.numpy as jnp
from jax import lax
from jax.experimental import pallas as pl
from jax.experimental.pallas import tpu as pltpu
from jax.experimental.pallas import tpu_sc as plsc

EMBED = 24


_SC_C = 1024
_SC_ZR = 128


@functools.cache
def _msgpass_sc(NNp: int, nchunks: int):
    C = _SC_C
    ZR = _SC_ZR
    R = NNp // 16
    cpt = nchunks // 16
    assert NNp % (16 * ZR) == 0 and nchunks % 16 == 0
    mesh = plsc.VectorSubcoreMesh(core_axis_name="c", subcore_axis_name="s")

    @functools.partial(
        pl.kernel,
        out_type=jax.ShapeDtypeStruct((2, NNp, 16), jnp.float32),
        mesh=mesh,
        compiler_params=pltpu.CompilerParams(
            needs_layout_passes=False, use_tc_tiling_on_sc=False),
        scratch_types=[
            pltpu.VMEM_SHARED((NNp, 16), jnp.float32),
            pltpu.VMEM((C,), jnp.int32),
            pltpu.VMEM((C,), jnp.int32),
            pltpu.VMEM((C,), jnp.float32),
            pltpu.VMEM((C, 16), jnp.float32),
            pltpu.VMEM((ZR, 16), jnp.float32),
            pltpu.SemaphoreType.DMA,
        ],
    )
    def k(m2, srch, dsth, ewh, out, agg_sh, src_v, dst_v, ew_v, rows_v, zbuf_v, sem):
        c = lax.axis_index("c")
        s = lax.axis_index("s")

        def zb(i, carry):
            zbuf_v[i] = jnp.zeros((16,), jnp.float32)
            return carry
        lax.fori_loop(0, ZR, zb, 0)

        def za(i, carry):
            pltpu.sync_copy(zbuf_v, agg_sh.at[pl.ds(s * R + i * ZR, ZR)])
            return carry
        lax.fori_loop(0, R // ZR, za, 0)
        plsc.subcore_barrier()

        def chunk_body(t, carry):
            base = (t * 16 + s) * C
            pltpu.sync_copy(srch.at[pl.ds(base, C)], src_v)
            pltpu.sync_copy(dsth.at[pl.ds(base, C)], dst_v)
            pltpu.sync_copy(ewh.at[pl.ds(base, C)], ew_v)

            @pl.when(c == 0)
            def _():
                pltpu.async_copy(m2.at[0].at[src_v], rows_v, sem).wait()

            @pl.when(c == 1)
            def _():
                pltpu.async_copy(m2.at[1].at[src_v], rows_v, sem).wait()

            def mul_body(gi, carry2):
                eb = gi * 16
                for j in range(16):
                    w = plsc.load_gather(ew_v, [jnp.full((16,), eb + j, jnp.int32)])
                    rows_v[eb + j] = rows_v[eb + j] * w
                return carry2
            lax.fori_loop(0, C // 16, mul_body, 0)

            pltpu.sync_copy(rows_v, agg_sh.at[dst_v], add=True)
            return carry
        lax.fori_loop(0, cpt, chunk_body, 0)
        plsc.subcore_barrier()

        def wb(i, carry):
            rs = s * R + i * ZR
            pltpu.sync_copy(agg_sh.at[pl.ds(rs, ZR)], zbuf_v)

            @pl.when(c == 0)
            def _():
                pltpu.sync_copy(zbuf_v, out.at[0].at[pl.ds(rs, ZR)])

            @pl.when(c == 1)
            def _():
                pltpu.sync_copy(zbuf_v, out.at[1].at[pl.ds(rs, ZR)])
            return carry
        lax.fori_loop(0, R // ZR, wb, 0)

    return k


def _pad_to(n: int, mult: int) -> int:
    return ((n + mult - 1) // mult) * mult


_DUMMY = 1024


def _sc_agg(m, srcp, dstp, ewp, NN):
    NNp = _pad_to(NN + _DUMMY, 16 * _SC_ZR)
    nchunks = ewp.shape[0] // _SC_C
    m2 = jnp.zeros((2, NNp, 16), jnp.float32)
    m2 = m2.at[0, :NN, :12].set(m[:, :12])
    m2 = m2.at[1, :NN, :12].set(m[:, 12:])
    agg2 = _msgpass_sc(NNp, nchunks)(m2, srcp, dstp, ewp)
    return jnp.concatenate([agg2[0, :NN, :12], agg2[1, :NN, :12]], axis=1)


def _gru(m, h, Wih, Whh, bih, bhh):
    gi = m @ Wih.T + bih
    gh = h @ Whh.T + bhh
    ir, iz, inn = jnp.split(gi, 3, axis=1)
    hr, hz, hn = jnp.split(gh, 3, axis=1)
    r = jax.nn.sigmoid(ir + hr)
    z = jax.nn.sigmoid(iz + hz)
    n = jnp.tanh(inn + r * hn)
    return (1.0 - z) * n + z * h


def _ggc(x, ei, ew, p, pre):
    src, dst = ei[0], ei[1]
    NN = x.shape[0]
    W = p[pre + '_W']
    for i in range(2):
        m = x @ W[i]
        agg = _sc_agg(m, src, dst, ew, NN)
        x = _gru(agg, x, p[pre + '_Wih'], p[pre + '_Whh'], p[pre + '_bih'], p[pre + '_bhh'])
    return x


def _topk_pool(x, ei, ea, batch, w, Bn, n_per, ratio):
    k = math.ceil(n_per * ratio)
    score = jnp.tanh((x @ w) / jnp.linalg.norm(w))
    sv, si = jax.lax.top_k(score.reshape(Bn, n_per), k)
    perm = (si + (jnp.arange(Bn) * n_per)[:, None]).reshape(-1)
    xn = x[perm] * score[perm][:, None]
    bn = batch[perm]
    mapping = jnp.full((Bn * n_per + _DUMMY,), -1, dtype=jnp.int32).at[perm].set(jnp.arange(Bn * k, dtype=jnp.int32))
    ns = mapping[ei[0]]
    nd = mapping[ei[1]]
    valid = (ns >= 0) & (nd >= 0)
    dummy = Bn * k + (jnp.arange(ns.shape[0], dtype=jnp.int32) & (_DUMMY - 1))
    ns = jnp.where(valid, ns, dummy)
    nd = jnp.where(valid, nd, dummy)
    ean = jnp.where(valid[:, None], ea, 0.0)
    return xn, jnp.stack([ns, nd]), ean, bn


def _set2set(x, Bn, n_per, p):
    xr = x.reshape(Bn, n_per, EMBED)
    h = jnp.zeros((Bn, EMBED), dtype=x.dtype)
    c = jnp.zeros((Bn, EMBED), dtype=x.dtype)
    q_star = jnp.zeros((Bn, 2 * EMBED), dtype=x.dtype)
    for _ in range(2):
        g = q_star @ p['lstm_Wih'].T + p['lstm_bih'] + h @ p['lstm_Whh'].T + p['lstm_bhh']
        ii, ff, gg, oo = jnp.split(g, 4, axis=1)
        ii = jax.nn.sigmoid(ii)
        ff = jax.nn.sigmoid(ff)
        gg = jnp.tanh(gg)
        oo = jax.nn.sigmoid(oo)
        c = ff * c + ii * gg
        h = oo * jnp.tanh(c)
        q = h
        e = (xr * q[:, None, :]).sum(-1)
        a = jax.nn.softmax(e, axis=1)
        r = (a[..., None] * xr).sum(1)
        q_star = jnp.concatenate([q, r], axis=1)
    return q_star


def kernel(x, edge_attr, y, edge_index, batch, params):
    Bn = y.shape[0]
    n0 = x.shape[0] // Bn
    indices = jnp.tile(jnp.arange(n0), Bn)
    E = edge_index.shape[1]
    N = x.shape[0]
    Ep = _pad_to(E, _SC_C * 16)
    pad_ids = N + (jnp.arange(Ep - E, dtype=jnp.int32) & (_DUMMY - 1))
    edge_index = jnp.concatenate(
        [edge_index, jnp.stack([pad_ids, pad_ids])], axis=1)
    edge_attr = jnp.pad(edge_attr, ((0, Ep - E), (0, 0)))
    x = jax.nn.relu(_ggc(x, edge_index, edge_attr[:, 0], params, 'conv1'))
    x, ei, ea, batch = _topk_pool(x, edge_index, edge_attr, batch, params['pool1_w'], Bn, n0, 0.8)
    n1 = x.shape[0] // Bn
    x = jax.nn.relu(_ggc(x, ei, ea[:, 0], params, 'conv2'))
    x, ei, ea, batch = _topk_pool(x, ei, ea, batch, params['pool2_w'], Bn, n1, 0.8)
    n2 = x.shape[0] // Bn
    x = jax.nn.relu(_ggc(x, ei, ea[:, 0], params, 'conv3'))
    x, ei, ea, batch = _topk_pool(x, ei, ea, batch, params['pool3_w'], Bn, n2, 0.3)
    n3 = x.shape[0] // Bn
    x = jax.nn.relu(_ggc(x, ei, ea[:, 0], params, 'conv4'))
    x = jax.nn.relu(_ggc(x, ei, ea[:, 0], params, 'conv5'))
    x = jax.nn.relu(_ggc(x, ei, ea[:, 0], params, 'conv6'))
    xr = x.reshape(Bn, n3, EMBED)
    gmp = xr.max(axis=1)
    gap = xr.mean(axis=1)
    s2s = _set2set(x, Bn, n3, params)
    x6 = jnp.concatenate([gmp, gap, s2s], axis=1)
    out = jax.nn.relu(x6 @ params['lin1_W'].T + params['lin1_b'])
    return out, indices
